# Initial kernel scaffold; baseline (speedup 1.0000x reference)
import functools

import jax
import jax.numpy as jnp
from jax import lax
from jax.experimental import pallas as pl
from jax.experimental.pallas import tpu as pltpu
from jax.experimental.pallas import tpu_sc as plsc

CHUNK = 128
NBUF = 4


@functools.lru_cache(maxsize=None)
def _make_gather(vocab: int, embed: int, n_rows: int):
  info = plsc.get_sparse_core_info()
  nw = info.num_cores * info.num_subcores
  nc = info.num_cores

  assert n_rows % (nw * CHUNK * NBUF * 2) == 0
  bpw = n_rows // nw
  nchunk = bpw // CHUNK
  ngroup = nchunk // NBUF
  assert ngroup % 2 == 0

  mesh = plsc.VectorSubcoreMesh(core_axis_name="c", subcore_axis_name="s")

  @functools.partial(
      pl.kernel,
      mesh=mesh,
      out_type=jax.ShapeDtypeStruct((n_rows, embed), jnp.float32),
      scratch_types=[
          pltpu.VMEM((nchunk, CHUNK), jnp.int32),
          pltpu.VMEM((2, NBUF, CHUNK, embed), jnp.float32),
          pltpu.SemaphoreType.DMA((2, NBUF)),
      ],
  )
  def gather_kernel(idx_hbm, table_hbm, out_hbm, idx_v, rows_v, gsem):
    wid = lax.axis_index("s") * nc + lax.axis_index("c")
    base = wid * bpw

    pltpu.sync_copy(idx_hbm.at[wid], idx_v)

    def issue_group(g, s):
      for b in range(NBUF):
        j = g * NBUF + b
        pltpu.async_copy(table_hbm.at[idx_v.at[j]], rows_v.at[s, b],
                         gsem.at[s, b])

    def drain_group(g, s):
      for b in range(NBUF):
        j = g * NBUF + b
        pltpu.make_async_copy(table_hbm.at[idx_v.at[j]], rows_v.at[s, b],
                              gsem.at[s, b]).wait()
        pltpu.sync_copy(rows_v.at[s, b],
                        out_hbm.at[pl.ds(base + j * CHUNK, CHUNK)])

    issue_group(0, 0)

    def body(gg):
      g = gg * 2
      issue_group(g + 1, 1)
      drain_group(g, 0)
      issue_group(g + 2, 0)
      drain_group(g + 1, 1)

    pl.loop(0, ngroup // 2 - 1)(body)

    g_last = ngroup - 2
    issue_group(g_last + 1, 1)
    drain_group(g_last, 0)
    drain_group(g_last + 1, 1)

  return gather_kernel, nw, nchunk


@jax.jit
def kernel(tensor, table):
  batch, hist = tensor.shape
  vocab, embed = table.shape
  n_rows = batch * hist
  fn, nw, nchunk = _make_gather(vocab, embed, n_rows)
  idx = tensor.reshape(nw, nchunk, CHUNK)
  out = fn(idx, table)
  return out.reshape(batch, hist, embed)


# speedup vs baseline: 1.8741x; 1.8741x over previous
import functools

import jax
import jax.numpy as jnp
from jax import lax
from jax.experimental import pallas as pl
from jax.experimental.pallas import tpu as pltpu
from jax.experimental.pallas import tpu_sc as plsc

CHUNK = 128
NBUF = 4


@functools.lru_cache(maxsize=None)
def _make_gather(vocab: int, embed: int, n_rows: int):
  info = plsc.get_sparse_core_info()
  nw = info.num_cores * info.num_subcores
  nc = info.num_cores

  assert n_rows % (nw * CHUNK * NBUF * 2) == 0
  bpw = n_rows // nw
  nchunk = bpw // CHUNK
  ngroup = nchunk // NBUF
  assert ngroup % 2 == 0

  mesh = plsc.VectorSubcoreMesh(core_axis_name="c", subcore_axis_name="s")

  @functools.partial(
      pl.kernel,
      mesh=mesh,
      compiler_params=pltpu.CompilerParams(use_tc_tiling_on_sc=False),
      out_type=jax.ShapeDtypeStruct((n_rows, embed), jnp.float32),
      scratch_types=[
          pltpu.VMEM((nchunk, CHUNK), jnp.int32),
          pltpu.VMEM((2, NBUF, CHUNK, embed), jnp.float32),
          pltpu.SemaphoreType.DMA((2, NBUF)),
      ],
  )
  def gather_kernel(idx_hbm, table_hbm, out_hbm, idx_v, rows_v, gsem):
    wid = lax.axis_index("s") * nc + lax.axis_index("c")
    base = wid * bpw

    pltpu.sync_copy(idx_hbm.at[wid], idx_v)

    def issue_group(g, s):
      for b in range(NBUF):
        j = g * NBUF + b
        pltpu.async_copy(table_hbm.at[idx_v.at[j]], rows_v.at[s, b],
                         gsem.at[s, b])

    def drain_group(g, s):
      for b in range(NBUF):
        j = g * NBUF + b
        pltpu.make_async_copy(table_hbm.at[idx_v.at[j]], rows_v.at[s, b],
                              gsem.at[s, b]).wait()
        pltpu.sync_copy(rows_v.at[s, b],
                        out_hbm.at[pl.ds(base + j * CHUNK, CHUNK)])

    issue_group(0, 0)

    def body(gg):
      g = gg * 2
      issue_group(g + 1, 1)
      drain_group(g, 0)
      issue_group(g + 2, 0)
      drain_group(g + 1, 1)

    pl.loop(0, ngroup // 2 - 1)(body)

    g_last = ngroup - 2
    issue_group(g_last + 1, 1)
    drain_group(g_last, 0)
    drain_group(g_last + 1, 1)

  return gather_kernel, nw, nchunk


@jax.jit
def kernel(tensor, table):
  batch, hist = tensor.shape
  vocab, embed = table.shape
  n_rows = batch * hist
  fn, nw, nchunk = _make_gather(vocab, embed, n_rows)
  idx = tensor.reshape(nw, nchunk, CHUNK)
  out = fn(idx, table)
  return out.reshape(batch, hist, embed)
